# baseline (speedup 1.0000x reference)
import functools

import jax
import jax.numpy as jnp
from jax import lax
from jax.experimental import pallas as pl
from jax.experimental.pallas import tpu as pltpu
from jax.experimental.pallas import tpu_sc as plsc


def _sc_gather(table, idx):
    NT = idx.shape[0]
    D = table.shape[1]
    info = plsc.get_sparse_core_info()
    NC, NS = info.num_cores, info.num_subcores
    NW = NC * NS
    per_w = NT // NW
    CH = 128
    n_ch = per_w // CH

    mesh = plsc.VectorSubcoreMesh(core_axis_name="c", subcore_axis_name="s")

    @functools.partial(
        pl.kernel,
        mesh=mesh,
        out_type=jax.ShapeDtypeStruct((NT, D), jnp.float32),
        scratch_types=[
            pltpu.VMEM((CH,), jnp.int32),
            pltpu.VMEM((CH, D), jnp.float32),
            pltpu.SemaphoreType.DMA,
        ],
    )
    def gk(idx_hbm, table_hbm, out_hbm, idx_v, rows_v, sem):
        wid = lax.axis_index("s") * NC + lax.axis_index("c")
        base = wid * per_w
        for j in range(n_ch):
            off = base + j * CH
            pltpu.sync_copy(idx_hbm.at[pl.ds(off, CH)], idx_v)
            pltpu.async_copy(table_hbm.at[idx_v], rows_v, sem).wait()
            pltpu.sync_copy(rows_v, out_hbm.at[pl.ds(off, CH)])

    return gk(idx, table)


def _tc_body(w_ref, tt_ref, pw_ref, pb_ref, pos_ref, tb_ref, g_ref, bt_ref,
             o_ref):
    x = jnp.dot(w_ref[0], pw_ref[...], preferred_element_type=jnp.float32)
    x = x + pb_ref[...] + pos_ref[...]
    tid = tt_ref[0, 0].astype(jnp.float32)
    x = x + tb_ref[0:1, :] + tid * (tb_ref[1:2, :] - tb_ref[0:1, :])
    mean = jnp.mean(x, axis=1, keepdims=True)
    xc = x - mean
    var = jnp.mean(xc * xc, axis=1, keepdims=True)
    inv = lax.rsqrt(var + 1e-5)
    o_ref[0] = (xc * inv) * g_ref[...] + bt_ref[...]


def _tc_body_alias(prev_ref, w_ref, tt_ref, pw_ref, pb_ref, pos_ref, tb_ref,
                   g_ref, bt_ref, o_ref):
    del prev_ref
    _tc_body(w_ref, tt_ref, pw_ref, pb_ref, pos_ref, tb_ref, g_ref, bt_ref,
             o_ref)


def kernel(input_ids, token_type_ids, word_table, proj_W, proj_b,
           pos_table, type_table, ln_gamma, ln_beta):
    B, S = input_ids.shape
    V, E = word_table.shape
    H = proj_W.shape[1]
    BS = 2048
    BH = B // 2

    idx = input_ids.reshape(-1).astype(jnp.int32)
    NTH = BH * S
    g0 = _sc_gather(word_table, idx[:NTH]).reshape(BH, S, E)
    g1 = _sc_gather(word_table, idx[NTH:]).reshape(BH, S, E)
    tt = token_type_ids.astype(jnp.int32).reshape(B, S // BS, BS, 1)

    common = (proj_W, proj_b.reshape(1, H), pos_table, type_table,
              ln_gamma.reshape(1, H), ln_beta.reshape(1, H))
    common_specs = [
        pl.BlockSpec((E, H), lambda s, b: (0, 0)),
        pl.BlockSpec((1, H), lambda s, b: (0, 0)),
        pl.BlockSpec((BS, H), lambda s, b: (s, 0)),
        pl.BlockSpec((2, H), lambda s, b: (0, 0)),
        pl.BlockSpec((1, H), lambda s, b: (0, 0)),
        pl.BlockSpec((1, H), lambda s, b: (0, 0)),
    ]
    out_shape = jax.ShapeDtypeStruct((B, S, H), jnp.float32)

    out0 = pl.pallas_call(
        _tc_body,
        grid=(S // BS, BH),
        in_specs=[
            pl.BlockSpec((1, BS, E), lambda s, b: (b, s, 0)),
            pl.BlockSpec((1, 1, BS, 1), lambda s, b: (b, s, 0, 0)),
        ] + common_specs,
        out_specs=pl.BlockSpec((1, BS, H), lambda s, b: (b, s, 0)),
        out_shape=out_shape,
    )(g0, tt[:BH], *common)

    out = pl.pallas_call(
        _tc_body_alias,
        grid=(S // BS, BH),
        in_specs=[
            pl.BlockSpec(memory_space=pl.ANY),
            pl.BlockSpec((1, BS, E), lambda s, b: (b, s, 0)),
            pl.BlockSpec((1, 1, BS, 1), lambda s, b: (b, s, 0, 0)),
        ] + common_specs,
        out_specs=pl.BlockSpec((1, BS, H), lambda s, b: (b + BH, s, 0)),
        out_shape=out_shape,
        input_output_aliases={0: 0},
    )(out0, g1, tt[BH:], *common)

    return out


# speedup vs baseline: 1.0587x; 1.0587x over previous
import functools

import jax
import jax.numpy as jnp
from jax import lax
from jax.experimental import pallas as pl
from jax.experimental.pallas import tpu as pltpu
from jax.experimental.pallas import tpu_sc as plsc


def _sc_gather(table, idx):
    NT = idx.shape[0]
    D = table.shape[1]
    info = plsc.get_sparse_core_info()
    NC, NS = info.num_cores, info.num_subcores
    NW = NC * NS
    per_w = NT // NW
    CH = 128
    n_ch = per_w // CH

    mesh = plsc.VectorSubcoreMesh(core_axis_name="c", subcore_axis_name="s")

    @functools.partial(
        pl.kernel,
        mesh=mesh,
        out_type=jax.ShapeDtypeStruct((NT, D), jnp.float32),
        scratch_types=[
            pltpu.VMEM((CH,), jnp.int32),
            pltpu.VMEM((CH, D), jnp.float32),
            pltpu.SemaphoreType.DMA,
        ],
    )
    def gk(idx_hbm, table_hbm, out_hbm, idx_v, rows_v, sem):
        wid = lax.axis_index("s") * NC + lax.axis_index("c")
        base = wid * per_w
        for j in range(n_ch):
            off = base + j * CH
            pltpu.sync_copy(idx_hbm.at[pl.ds(off, CH)], idx_v)
            pltpu.async_copy(table_hbm.at[idx_v], rows_v, sem).wait()
            pltpu.sync_copy(rows_v, out_hbm.at[pl.ds(off, CH)])

    return gk(idx, table)


def _tc_body(w_ref, tt_ref, pw_ref, pb_ref, pos_ref, tb_ref, g_ref, bt_ref,
             o_ref):
    x = jnp.dot(w_ref[0], pw_ref[...], preferred_element_type=jnp.float32)
    x = x + pb_ref[...] + pos_ref[...]
    tid = tt_ref[0, 0].astype(jnp.float32)
    x = x + tb_ref[0:1, :] + tid * (tb_ref[1:2, :] - tb_ref[0:1, :])
    mean = jnp.mean(x, axis=1, keepdims=True)
    xc = x - mean
    var = jnp.mean(xc * xc, axis=1, keepdims=True)
    inv = lax.rsqrt(var + 1e-5)
    o_ref[0] = (xc * inv) * g_ref[...] + bt_ref[...]


def _tc_body_alias(prev_ref, w_ref, tt_ref, pw_ref, pb_ref, pos_ref, tb_ref,
                   g_ref, bt_ref, o_ref):
    del prev_ref
    _tc_body(w_ref, tt_ref, pw_ref, pb_ref, pos_ref, tb_ref, g_ref, bt_ref,
             o_ref)


def kernel(input_ids, token_type_ids, word_table, proj_W, proj_b,
           pos_table, type_table, ln_gamma, ln_beta):
    B, S = input_ids.shape
    V, E = word_table.shape
    H = proj_W.shape[1]
    BS = 1024
    S2 = S // 2
    NB2 = S2 // BS

    ids32 = input_ids.astype(jnp.int32)
    g0 = _sc_gather(word_table, ids32[:, :S2].reshape(-1)).reshape(B, S2, E)
    g1 = _sc_gather(word_table, ids32[:, S2:].reshape(-1)).reshape(B, S2, E)
    tt = token_type_ids.astype(jnp.int32).reshape(B, S // BS, BS, 1)

    common = (proj_W, proj_b.reshape(1, H), pos_table, type_table,
              ln_gamma.reshape(1, H), ln_beta.reshape(1, H))
    out_shape = jax.ShapeDtypeStruct((B, S, H), jnp.float32)

    def half_specs(off):
        return [
            pl.BlockSpec((E, H), lambda s, b: (0, 0)),
            pl.BlockSpec((1, H), lambda s, b: (0, 0)),
            pl.BlockSpec((BS, H), lambda s, b: (s + off, 0)),
            pl.BlockSpec((2, H), lambda s, b: (0, 0)),
            pl.BlockSpec((1, H), lambda s, b: (0, 0)),
            pl.BlockSpec((1, H), lambda s, b: (0, 0)),
        ]

    out0 = pl.pallas_call(
        _tc_body,
        grid=(NB2, B),
        in_specs=[
            pl.BlockSpec((1, BS, E), lambda s, b: (b, s, 0)),
            pl.BlockSpec((1, 1, BS, 1), lambda s, b: (b, s, 0, 0)),
        ] + half_specs(0),
        out_specs=pl.BlockSpec((1, BS, H), lambda s, b: (b, s, 0)),
        out_shape=out_shape,
    )(g0, tt, *common)

    out = pl.pallas_call(
        _tc_body_alias,
        grid=(NB2, B),
        in_specs=[
            pl.BlockSpec(memory_space=pl.ANY),
            pl.BlockSpec((1, BS, E), lambda s, b: (b, s, 0)),
            pl.BlockSpec((1, 1, BS, 1), lambda s, b: (b, s + NB2, 0, 0)),
        ] + half_specs(NB2),
        out_specs=pl.BlockSpec((1, BS, H), lambda s, b: (b, s + NB2, 0)),
        out_shape=out_shape,
        input_output_aliases={0: 0},
    )(out0, g1, tt, *common)

    return out


# speedup vs baseline: 1.1120x; 1.0503x over previous
import functools

import jax
import jax.numpy as jnp
from jax import lax
from jax.experimental import pallas as pl
from jax.experimental.pallas import tpu as pltpu
from jax.experimental.pallas import tpu_sc as plsc


def _sc_gather(table, idx):
    NT = idx.shape[0]
    D = table.shape[1]
    info = plsc.get_sparse_core_info()
    NC, NS = info.num_cores, info.num_subcores
    NW = NC * NS
    per_w = NT // NW
    CH = 128
    n_ch = per_w // CH

    mesh = plsc.VectorSubcoreMesh(core_axis_name="c", subcore_axis_name="s")

    @functools.partial(
        pl.kernel,
        mesh=mesh,
        out_type=jax.ShapeDtypeStruct((NT, D), jnp.float32),
        scratch_types=[
            pltpu.VMEM((CH,), jnp.int32),
            pltpu.VMEM((CH, D), jnp.float32),
            pltpu.SemaphoreType.DMA,
        ],
    )
    def gk(idx_hbm, table_hbm, out_hbm, idx_v, rows_v, sem):
        wid = lax.axis_index("s") * NC + lax.axis_index("c")
        base = wid * per_w
        for j in range(n_ch):
            off = base + j * CH
            pltpu.sync_copy(idx_hbm.at[pl.ds(off, CH)], idx_v)
            pltpu.async_copy(table_hbm.at[idx_v], rows_v, sem).wait()
            pltpu.sync_copy(rows_v, out_hbm.at[pl.ds(off, CH)])

    return gk(idx, table)


def _tc_body(w_ref, tt_ref, pw_ref, pb_ref, pos_ref, tb_ref, g_ref, bt_ref,
             o_ref):
    x = jnp.dot(w_ref[0], pw_ref[...], preferred_element_type=jnp.float32)
    x = x + pb_ref[...] + pos_ref[...]
    tid = tt_ref[0, 0].astype(jnp.float32)
    x = x + tb_ref[0:1, :] + tid * (tb_ref[1:2, :] - tb_ref[0:1, :])
    mean = jnp.mean(x, axis=1, keepdims=True)
    xc = x - mean
    var = jnp.mean(xc * xc, axis=1, keepdims=True)
    inv = lax.rsqrt(var + 1e-5)
    o_ref[0] = (xc * inv) * g_ref[...] + bt_ref[...]


def _tc_body_alias(prev_ref, w_ref, tt_ref, pw_ref, pb_ref, pos_ref, tb_ref,
                   g_ref, bt_ref, o_ref):
    del prev_ref
    _tc_body(w_ref, tt_ref, pw_ref, pb_ref, pos_ref, tb_ref, g_ref, bt_ref,
             o_ref)


def kernel(input_ids, token_type_ids, word_table, proj_W, proj_b,
           pos_table, type_table, ln_gamma, ln_beta):
    B, S = input_ids.shape
    V, E = word_table.shape
    H = proj_W.shape[1]
    BS = 2048

    idx = input_ids.reshape(-1).astype(jnp.int32)
    gathered = _sc_gather(word_table, idx).reshape(B, S, E)
    tt = token_type_ids.astype(jnp.int32).reshape(B, S // BS, BS, 1)

    out = pl.pallas_call(
        _tc_body,
        grid=(S // BS, B),
        in_specs=[
            pl.BlockSpec((1, BS, E), lambda s, b: (b, s, 0)),
            pl.BlockSpec((1, 1, BS, 1), lambda s, b: (b, s, 0, 0)),
            pl.BlockSpec((E, H), lambda s, b: (0, 0)),
            pl.BlockSpec((1, H), lambda s, b: (0, 0)),
            pl.BlockSpec((BS, H), lambda s, b: (s, 0)),
            pl.BlockSpec((2, H), lambda s, b: (0, 0)),
            pl.BlockSpec((1, H), lambda s, b: (0, 0)),
            pl.BlockSpec((1, H), lambda s, b: (0, 0)),
        ],
        out_specs=pl.BlockSpec((1, BS, H), lambda s, b: (b, s, 0)),
        out_shape=jax.ShapeDtypeStruct((B, S, H), jnp.float32),
    )(gathered, tt, proj_W, proj_b.reshape(1, H), pos_table,
      type_table, ln_gamma.reshape(1, H), ln_beta.reshape(1, H))

    return out


# speedup vs baseline: 1.1443x; 1.0291x over previous
import functools

import jax
import jax.numpy as jnp
from jax import lax
from jax.experimental import pallas as pl
from jax.experimental.pallas import tpu as pltpu
from jax.experimental.pallas import tpu_sc as plsc


def _sc_gather(table, idx):
    NT = idx.shape[0]
    D = table.shape[1]
    info = plsc.get_sparse_core_info()
    NC, NS = info.num_cores, info.num_subcores
    NW = NC * NS
    per_w = NT // NW
    CH = 128
    n_ch = per_w // CH

    mesh = plsc.VectorSubcoreMesh(core_axis_name="c", subcore_axis_name="s")

    @functools.partial(
        pl.kernel,
        mesh=mesh,
        out_type=jax.ShapeDtypeStruct((NT, D), jnp.float32),
        scratch_types=[
            pltpu.VMEM((per_w,), jnp.int32),
            pltpu.VMEM((CH, D), jnp.float32),
            pltpu.VMEM((CH, D), jnp.float32),
            pltpu.SemaphoreType.DMA,
            pltpu.SemaphoreType.DMA,
            pltpu.SemaphoreType.DMA,
            pltpu.SemaphoreType.DMA,
        ],
    )
    def gk(idx_hbm, table_hbm, out_hbm, idx_v, r0, r1, sg0, sg1, sw0, sw1):
        wid = lax.axis_index("s") * NC + lax.axis_index("c")
        base = wid * per_w
        pltpu.sync_copy(idx_hbm.at[pl.ds(base, per_w)], idx_v)
        c0 = pltpu.async_copy(table_hbm.at[idx_v.at[pl.ds(0, CH)]], r0, sg0)
        c1 = pltpu.async_copy(table_hbm.at[idx_v.at[pl.ds(CH, CH)]], r1, sg1)
        c0.wait()
        w0 = pltpu.async_copy(r0, out_hbm.at[pl.ds(base, CH)], sw0)
        c1.wait()
        w1 = pltpu.async_copy(r1, out_hbm.at[pl.ds(base + CH, CH)], sw1)
        w0.wait()
        w1.wait()

    return gk(idx, table)


def _tc_body(w_ref, tt_ref, pw_ref, pb_ref, pos_ref, tb_ref, g_ref, bt_ref,
             o_ref):
    x = jnp.dot(w_ref[0], pw_ref[...], preferred_element_type=jnp.float32)
    x = x + pb_ref[...] + pos_ref[...]
    tid = tt_ref[0, 0].astype(jnp.float32)
    x = x + tb_ref[0:1, :] + tid * (tb_ref[1:2, :] - tb_ref[0:1, :])
    mean = jnp.mean(x, axis=1, keepdims=True)
    xc = x - mean
    var = jnp.mean(xc * xc, axis=1, keepdims=True)
    inv = lax.rsqrt(var + 1e-5)
    o_ref[0] = (xc * inv) * g_ref[...] + bt_ref[...]


def _tc_body_alias(prev_ref, w_ref, tt_ref, pw_ref, pb_ref, pos_ref, tb_ref,
                   g_ref, bt_ref, o_ref):
    del prev_ref
    _tc_body(w_ref, tt_ref, pw_ref, pb_ref, pos_ref, tb_ref, g_ref, bt_ref,
             o_ref)


def kernel(input_ids, token_type_ids, word_table, proj_W, proj_b,
           pos_table, type_table, ln_gamma, ln_beta):
    B, S = input_ids.shape
    V, E = word_table.shape
    H = proj_W.shape[1]
    BS = 2048

    idx = input_ids.reshape(-1).astype(jnp.int32)
    gathered = _sc_gather(word_table, idx).reshape(B, S, E)
    tt = token_type_ids.astype(jnp.int32).reshape(B, S // BS, BS, 1)

    out = pl.pallas_call(
        _tc_body,
        grid=(S // BS, B),
        in_specs=[
            pl.BlockSpec((1, BS, E), lambda s, b: (b, s, 0)),
            pl.BlockSpec((1, 1, BS, 1), lambda s, b: (b, s, 0, 0)),
            pl.BlockSpec((E, H), lambda s, b: (0, 0)),
            pl.BlockSpec((1, H), lambda s, b: (0, 0)),
            pl.BlockSpec((BS, H), lambda s, b: (s, 0)),
            pl.BlockSpec((2, H), lambda s, b: (0, 0)),
            pl.BlockSpec((1, H), lambda s, b: (0, 0)),
            pl.BlockSpec((1, H), lambda s, b: (0, 0)),
        ],
        out_specs=pl.BlockSpec((1, BS, H), lambda s, b: (b, s, 0)),
        out_shape=jax.ShapeDtypeStruct((B, S, H), jnp.float32),
    )(gathered, tt, proj_W, proj_b.reshape(1, H), pos_table,
      type_table, ln_gamma.reshape(1, H), ln_beta.reshape(1, H))

    return out


# speedup vs baseline: 1.1445x; 1.0002x over previous
import functools

import jax
import jax.numpy as jnp
from jax import lax
from jax.experimental import pallas as pl
from jax.experimental.pallas import tpu as pltpu
from jax.experimental.pallas import tpu_sc as plsc


def _sc_gather(table, idx):
    NT = idx.shape[0]
    D = table.shape[1]
    info = plsc.get_sparse_core_info()
    NC, NS = info.num_cores, info.num_subcores
    NW = NC * NS
    per_w = NT // NW
    CH = 64
    n_ch = per_w // CH

    mesh = plsc.VectorSubcoreMesh(core_axis_name="c", subcore_axis_name="s")

    @functools.partial(
        pl.kernel,
        mesh=mesh,
        out_type=jax.ShapeDtypeStruct((NT, D), jnp.float32),
        scratch_types=(
            [pltpu.VMEM((per_w,), jnp.int32)]
            + [pltpu.VMEM((CH, D), jnp.float32)] * n_ch
            + [pltpu.SemaphoreType.DMA] * (2 * n_ch)
        ),
    )
    def gk(idx_hbm, table_hbm, out_hbm, idx_v, *scr):
        rows = scr[:n_ch]
        sg = scr[n_ch:2 * n_ch]
        sw = scr[2 * n_ch:]
        wid = lax.axis_index("s") * NC + lax.axis_index("c")
        base = wid * per_w
        pltpu.sync_copy(idx_hbm.at[pl.ds(base, per_w)], idx_v)
        gathers = [
            pltpu.async_copy(table_hbm.at[idx_v.at[pl.ds(j * CH, CH)]],
                             rows[j], sg[j])
            for j in range(n_ch)
        ]
        writes = []
        for j in range(n_ch):
            gathers[j].wait()
            writes.append(pltpu.async_copy(
                rows[j], out_hbm.at[pl.ds(base + j * CH, CH)], sw[j]))
        for w in writes:
            w.wait()

    return gk(idx, table)


def _tc_body(w_ref, tt_ref, pw_ref, pb_ref, pos_ref, tb_ref, g_ref, bt_ref,
             o_ref):
    x = jnp.dot(w_ref[0], pw_ref[...], preferred_element_type=jnp.float32)
    x = x + pb_ref[...] + pos_ref[...]
    tid = tt_ref[0, 0].astype(jnp.float32)
    x = x + tb_ref[0:1, :] + tid * (tb_ref[1:2, :] - tb_ref[0:1, :])
    mean = jnp.mean(x, axis=1, keepdims=True)
    xc = x - mean
    var = jnp.mean(xc * xc, axis=1, keepdims=True)
    inv = lax.rsqrt(var + 1e-5)
    o_ref[0] = (xc * inv) * g_ref[...] + bt_ref[...]


def _tc_body_alias(prev_ref, w_ref, tt_ref, pw_ref, pb_ref, pos_ref, tb_ref,
                   g_ref, bt_ref, o_ref):
    del prev_ref
    _tc_body(w_ref, tt_ref, pw_ref, pb_ref, pos_ref, tb_ref, g_ref, bt_ref,
             o_ref)


def kernel(input_ids, token_type_ids, word_table, proj_W, proj_b,
           pos_table, type_table, ln_gamma, ln_beta):
    B, S = input_ids.shape
    V, E = word_table.shape
    H = proj_W.shape[1]
    BS = 2048

    idx = input_ids.reshape(-1).astype(jnp.int32)
    gathered = _sc_gather(word_table, idx).reshape(B, S, E)
    tt = token_type_ids.astype(jnp.int32).reshape(B, S // BS, BS, 1)

    out = pl.pallas_call(
        _tc_body,
        grid=(S // BS, B),
        in_specs=[
            pl.BlockSpec((1, BS, E), lambda s, b: (b, s, 0)),
            pl.BlockSpec((1, 1, BS, 1), lambda s, b: (b, s, 0, 0)),
            pl.BlockSpec((E, H), lambda s, b: (0, 0)),
            pl.BlockSpec((1, H), lambda s, b: (0, 0)),
            pl.BlockSpec((BS, H), lambda s, b: (s, 0)),
            pl.BlockSpec((2, H), lambda s, b: (0, 0)),
            pl.BlockSpec((1, H), lambda s, b: (0, 0)),
            pl.BlockSpec((1, H), lambda s, b: (0, 0)),
        ],
        out_specs=pl.BlockSpec((1, BS, H), lambda s, b: (b, s, 0)),
        out_shape=jax.ShapeDtypeStruct((B, S, H), jnp.float32),
    )(gathered, tt, proj_W, proj_b.reshape(1, H), pos_table,
      type_table, ln_gamma.reshape(1, H), ln_beta.reshape(1, H))

    return out


# speedup vs baseline: 1.1578x; 1.0117x over previous
import functools

import jax
import jax.numpy as jnp
from jax import lax
from jax.experimental import pallas as pl
from jax.experimental.pallas import tpu as pltpu
from jax.experimental.pallas import tpu_sc as plsc


def _sc_gather(table, idx):
    NT = idx.shape[0]
    D = table.shape[1]
    info = plsc.get_sparse_core_info()
    NC, NS = info.num_cores, info.num_subcores
    NW = NC * NS
    per_w = NT // NW
    CH = 64
    n_ch = per_w // CH

    mesh = plsc.VectorSubcoreMesh(core_axis_name="c", subcore_axis_name="s")

    @functools.partial(
        pl.kernel,
        mesh=mesh,
        out_type=jax.ShapeDtypeStruct((NT, D), jnp.float32),
        scratch_types=(
            [pltpu.VMEM((per_w,), jnp.int32)]
            + [pltpu.VMEM((CH, D), jnp.float32)] * n_ch
            + [pltpu.SemaphoreType.DMA] * (2 * n_ch)
        ),
    )
    def gk(idx_hbm, table_hbm, out_hbm, idx_v, *scr):
        rows = scr[:n_ch]
        sg = scr[n_ch:2 * n_ch]
        sw = scr[2 * n_ch:]
        wid = lax.axis_index("s") * NC + lax.axis_index("c")
        base = wid * per_w
        pltpu.sync_copy(idx_hbm.at[pl.ds(base, per_w)], idx_v)
        gathers = [
            pltpu.async_copy(table_hbm.at[idx_v.at[pl.ds(j * CH, CH)]],
                             rows[j], sg[j])
            for j in range(n_ch)
        ]
        writes = []
        for j in range(n_ch):
            gathers[j].wait()
            writes.append(pltpu.async_copy(
                rows[j], out_hbm.at[pl.ds(base + j * CH, CH)], sw[j]))
        for w in writes:
            w.wait()

    return gk(idx, table)


def _tc_body(w_ref, tt_ref, pw_ref, pb_ref, pos_ref, tb_ref, g_ref, bt_ref,
             o_ref):
    x = jnp.dot(w_ref[0], pw_ref[...], preferred_element_type=jnp.float32)
    x = x + pb_ref[...] + pos_ref[...]
    tid = tt_ref[0, 0].astype(jnp.float32)
    x = x + tb_ref[0:1, :] + tid * (tb_ref[1:2, :] - tb_ref[0:1, :])
    mean = jnp.mean(x, axis=1, keepdims=True)
    xc = x - mean
    var = jnp.mean(xc * xc, axis=1, keepdims=True)
    inv = lax.rsqrt(var + 1e-5)
    o_ref[0] = (xc * inv) * g_ref[...] + bt_ref[...]


def _tc_body_alias(prev_ref, w_ref, tt_ref, pw_ref, pb_ref, pos_ref, tb_ref,
                   g_ref, bt_ref, o_ref):
    del prev_ref
    _tc_body(w_ref, tt_ref, pw_ref, pb_ref, pos_ref, tb_ref, g_ref, bt_ref,
             o_ref)


def kernel(input_ids, token_type_ids, word_table, proj_W, proj_b,
           pos_table, type_table, ln_gamma, ln_beta):
    B, S = input_ids.shape
    V, E = word_table.shape
    H = proj_W.shape[1]
    BS = 2048

    idx = input_ids.reshape(-1).astype(jnp.int32)
    gathered = _sc_gather(word_table, idx).reshape(B, S, E)
    tt = token_type_ids.astype(jnp.int8).reshape(B, S // BS, BS, 1)

    out = pl.pallas_call(
        _tc_body,
        grid=(S // BS, B),
        in_specs=[
            pl.BlockSpec((1, BS, E), lambda s, b: (b, s, 0)),
            pl.BlockSpec((1, 1, BS, 1), lambda s, b: (b, s, 0, 0)),
            pl.BlockSpec((E, H), lambda s, b: (0, 0)),
            pl.BlockSpec((1, H), lambda s, b: (0, 0)),
            pl.BlockSpec((BS, H), lambda s, b: (s, 0)),
            pl.BlockSpec((2, H), lambda s, b: (0, 0)),
            pl.BlockSpec((1, H), lambda s, b: (0, 0)),
            pl.BlockSpec((1, H), lambda s, b: (0, 0)),
        ],
        out_specs=pl.BlockSpec((1, BS, H), lambda s, b: (b, s, 0)),
        out_shape=jax.ShapeDtypeStruct((B, S, H), jnp.float32),
    )(gathered, tt, proj_W, proj_b.reshape(1, H), pos_table,
      type_table, ln_gamma.reshape(1, H), ln_beta.reshape(1, H))

    return out
